# baseline (speedup 1.0000x reference)
import functools
import math

import jax
import jax.numpy as jnp
from jax import lax
from jax.experimental import pallas as pl
from jax.experimental.pallas import tpu as pltpu
from jax.experimental.pallas import tpu_sc as plsc

VOCAB = 1000000
DIM = 64
B = 16384
L = 20

NC = 2
NS = 16
NW = NC * NS

SENT_PER_W = B // NW
TOK_PER_W = SENT_PER_W * L
SENT_PER_CHUNK = 4
TOK_PER_CHUNK = SENT_PER_CHUNK * L
N_CHUNKS = SENT_PER_W // SENT_PER_CHUNK

INV_SQRT_L = 1.0 / math.sqrt(float(L))


def _sc_body(ids_hbm, table_hbm, out_hbm, idx_v, rows_v, out_v, sem):
  wid = lax.axis_index("s") * NC + lax.axis_index("c")

  pltpu.sync_copy(ids_hbm.at[wid], idx_v)

  def chunk_body(j):
    pltpu.async_copy(table_hbm.at[idx_v.at[j]], rows_v, sem).wait()
    for s in range(SENT_PER_CHUNK):
      for d in range(DIM // 16):
        acc = rows_v[s * L, pl.ds(d * 16, 16)]
        for l in range(1, L):
          acc = acc + rows_v[s * L + l, pl.ds(d * 16, 16)]
        out_v[j * SENT_PER_CHUNK + s, pl.ds(d * 16, 16)] = acc * INV_SQRT_L

  pl.loop(0, N_CHUNKS)(chunk_body)

  pltpu.sync_copy(out_v, out_hbm.at[pl.ds(wid * SENT_PER_W, SENT_PER_W)])


@jax.jit
def _pooled_embedding(ids, table):
  mesh = plsc.VectorSubcoreMesh(core_axis_name="c", subcore_axis_name="s")
  kern = functools.partial(
      pl.kernel,
      mesh=mesh,
      out_type=jax.ShapeDtypeStruct((B, DIM), jnp.float32),
      scratch_types=[
          pltpu.VMEM((N_CHUNKS, TOK_PER_CHUNK), jnp.int32),
          pltpu.VMEM((TOK_PER_CHUNK, DIM), jnp.float32),
          pltpu.VMEM((SENT_PER_W, DIM), jnp.float32),
          pltpu.SemaphoreType.DMA,
      ],
      compiler_params=pltpu.CompilerParams(use_tc_tiling_on_sc=False),
  )(_sc_body)
  return kern(ids, table)


def kernel(token_ids, embedding_table):
  ids = token_ids.reshape(NW, N_CHUNKS, TOK_PER_CHUNK)
  return _pooled_embedding(ids, embedding_table)


# speedup vs baseline: 1.0296x; 1.0296x over previous
import functools
import math

import jax
import jax.numpy as jnp
from jax import lax
from jax.experimental import pallas as pl
from jax.experimental.pallas import tpu as pltpu
from jax.experimental.pallas import tpu_sc as plsc

VOCAB = 1000000
DIM = 64
B = 16384
L = 20

NC = 2
NS = 16
NW = NC * NS

SENT_PER_W = B // NW
TOK_PER_W = SENT_PER_W * L
SENT_PER_CHUNK = 4
TOK_PER_CHUNK = SENT_PER_CHUNK * L
N_CHUNKS = SENT_PER_W // SENT_PER_CHUNK

INV_SQRT_L = 1.0 / math.sqrt(float(L))


NBUF = 4


def _sc_body(ids_hbm, table_hbm, out_hbm, idx_v, rows_v, out_v, sems):
  wid = lax.axis_index("s") * NC + lax.axis_index("c")

  pltpu.sync_copy(ids_hbm.at[wid], idx_v)

  def start_gather(j, b):
    pltpu.async_copy(table_hbm.at[idx_v.at[j]], rows_v.at[b], sems.at[b])

  def accumulate(j, b):
    for s in range(SENT_PER_CHUNK):
      for d in range(DIM // 16):
        acc = rows_v[b, s * L, pl.ds(d * 16, 16)]
        for l in range(1, L):
          acc = acc + rows_v[b, s * L + l, pl.ds(d * 16, 16)]
        out_v[j * SENT_PER_CHUNK + s, pl.ds(d * 16, 16)] = acc * INV_SQRT_L

  def wait(b):
    pltpu.make_async_copy(
        table_hbm.at[pl.ds(0, TOK_PER_CHUNK)], rows_v.at[b], sems.at[b]
    ).wait()

  for b in range(NBUF):
    start_gather(b, b)

  def ring_body(j):
    for b in range(NBUF):
      wait(b)
      accumulate(j + b, b)
      start_gather(j + b + NBUF, b)

  pl.loop(0, N_CHUNKS - NBUF, step=NBUF)(ring_body)

  for b in range(NBUF):
    wait(b)
    accumulate(N_CHUNKS - NBUF + b, b)

  pltpu.sync_copy(out_v, out_hbm.at[pl.ds(wid * SENT_PER_W, SENT_PER_W)])


@jax.jit
def _pooled_embedding(ids, table):
  mesh = plsc.VectorSubcoreMesh(core_axis_name="c", subcore_axis_name="s")
  kern = functools.partial(
      pl.kernel,
      mesh=mesh,
      out_type=jax.ShapeDtypeStruct((B, DIM), jnp.float32),
      scratch_types=[
          pltpu.VMEM((N_CHUNKS, TOK_PER_CHUNK), jnp.int32),
          pltpu.VMEM((NBUF, TOK_PER_CHUNK, DIM), jnp.float32),
          pltpu.VMEM((SENT_PER_W, DIM), jnp.float32),
          pltpu.SemaphoreType.DMA((NBUF,)),
      ],
      compiler_params=pltpu.CompilerParams(use_tc_tiling_on_sc=False),
  )(_sc_body)
  return kern(ids, table)


def kernel(token_ids, embedding_table):
  ids = token_ids.reshape(NW, N_CHUNKS, TOK_PER_CHUNK)
  return _pooled_embedding(ids, embedding_table)
